# baseline (speedup 1.0000x reference)
import jax
import jax.numpy as jnp
from jax.experimental import pallas as pl
from jax.experimental.pallas import tpu as pltpu

_B, _N, _DIN, _H, _OUT = 4, 128, 128, 128, 128
_NL, _DE = 4, 4
_NP = _N // 2
_EP = _N * _NP
_HP = 2 * _H

_F32 = jnp.float32
_BF = jnp.bfloat16


def _bd2(w):
    z = jnp.zeros(w.shape, _BF)
    wb = w.astype(_BF)
    return jnp.concatenate(
        [jnp.concatenate([wb, z], axis=1), jnp.concatenate([z, wb], axis=1)], axis=0)


def _gnn_body(x_ref, ea_ref, w1i_ref, b1i_ref, w2i_ref, b2i_ref, *rest):
    h_ref = rest[-1]
    wl_refs = rest[:-1]

    xb = x_ref[0].astype(_BF)
    h1 = jnp.maximum(
        jnp.dot(xb, w1i_ref[...].astype(_BF), preferred_element_type=_F32)
        + b1i_ref[...], 0.0)
    h = (jnp.dot(h1.astype(_BF), w2i_ref[...].astype(_BF), preferred_element_type=_F32)
         + b2i_ref[...])

    ea = ea_ref[0]
    for l in range(_NL):
        (we1_ref, be1_ref, we2_ref, be2_ref,
         wn1_ref, bn1_ref, wn2_ref, bn2_ref) = wl_refs[8 * l:8 * (l + 1)]
        we1 = we1_ref[...]
        hb = h.astype(_BF)
        a = (jnp.dot(hb, we1[:_H].astype(_BF), preferred_element_type=_F32)
             + be1_ref[...])
        ap = jnp.concatenate([a, a], axis=1)
        c = jnp.dot(hb, we1[_H:2 * _H].astype(_BF), preferred_element_type=_F32)
        cp = jnp.concatenate([c[:_NP], c[_NP:]], axis=1)
        w_aug = jnp.concatenate(
            [_bd2(we1[2 * _H:]), cp.astype(_BF), ap.astype(_BF)], axis=0)
        pre = jnp.dot(ea, w_aug, preferred_element_type=_F32)
        m1 = jnp.maximum(pre, 0.0).astype(_BF)
        be2 = be2_ref[...]
        m2 = jnp.maximum(
            jnp.dot(m1, _bd2(we2_ref[...]), preferred_element_type=_F32)
            + jnp.concatenate([be2, be2], axis=1), 0.0)
        ms = m2.reshape(_N, _NP, _HP).sum(axis=1)
        mi = ms[:, :_H] + ms[:, _H:]
        wn1 = wn1_ref[...]
        u = jnp.maximum(
            jnp.dot(hb, wn1[:_H].astype(_BF), preferred_element_type=_F32)
            + jnp.dot(mi.astype(_BF), wn1[_H:].astype(_BF), preferred_element_type=_F32)
            + bn1_ref[...], 0.0)
        h = (h + jnp.dot(u.astype(_BF), wn2_ref[...].astype(_BF),
                         preferred_element_type=_F32)
             + bn2_ref[...])

    h_ref[0] = h


def _out_body(hf_ref, w1_ref, b1_ref, w2_ref, b2_ref, o_ref):
    h1 = jnp.maximum(
        jnp.dot(hf_ref[...], w1_ref[...].astype(_BF), preferred_element_type=_F32)
        + b1_ref[...], 0.0)
    o_ref[...] = (jnp.dot(h1.astype(_BF), w2_ref[...].astype(_BF),
                          preferred_element_type=_F32)
                  + b2_ref[...])


def kernel(x, edge_attr, key_unused, params):
    p = params
    wl = []
    for l in range(_NL):
        wl += [p['l%d_We1' % l], p['l%d_be1' % l].reshape(1, _H),
               p['l%d_We2' % l], p['l%d_be2' % l].reshape(1, _H),
               p['l%d_Wn1' % l], p['l%d_bn1' % l].reshape(1, _H),
               p['l%d_Wn2' % l], p['l%d_bn2' % l].reshape(1, _H)]
    eap = (edge_attr.astype(_BF)
           .reshape(_B, _N, 2, _NP, _DE)
           .transpose(0, 1, 3, 2, 4)
           .reshape(_B, _EP, 2 * _DE))
    ohk = jnp.tile(jnp.eye(_NP, dtype=_BF), (_N, 1))
    ohi = jnp.repeat(jnp.eye(_N, dtype=_BF), _NP, axis=0)
    oh = jnp.broadcast_to(
        jnp.concatenate([ohk, ohi], axis=1)[None], (_B, _EP, _NP + _N))
    eap = jnp.concatenate([eap, oh], axis=2)

    _cw = lambda b: (0, 0)
    h = pl.pallas_call(
        _gnn_body,
        grid=(_B,),
        in_specs=[
            pl.BlockSpec((1, _N, _DIN), lambda b: (b, 0, 0)),
            pl.BlockSpec((1, _EP, 2 * _DE + _NP + _N), lambda b: (b, 0, 0)),
            pl.BlockSpec((_DIN, _H), _cw),
            pl.BlockSpec((1, _H), _cw),
            pl.BlockSpec((_H, _H), _cw),
            pl.BlockSpec((1, _H), _cw),
        ] + [pl.BlockSpec(w.shape, _cw) for w in wl],
        out_specs=pl.BlockSpec((1, _N, _H), lambda b: (b, 0, 0)),
        out_shape=jax.ShapeDtypeStruct((_B, _N, _H), _F32),
        compiler_params=pltpu.CompilerParams(
            dimension_semantics=("arbitrary",)),
    )(x, eap, p['emb_in_W1'], p['emb_in_b1'].reshape(1, _H),
      p['emb_in_W2'], p['emb_in_b2'].reshape(1, _H), *wl)

    hf = h.reshape(_B, _N * _H).astype(_BF)
    out = pl.pallas_call(
        _out_body,
        out_shape=jax.ShapeDtypeStruct((_B, _OUT), _F32),
    )(hf, p['emb_out_W1'], p['emb_out_b1'].reshape(1, _H),
      p['emb_out_W2'], p['emb_out_b2'].reshape(1, _OUT))
    return out


# speedup vs baseline: 1.2340x; 1.2340x over previous
import jax
import jax.numpy as jnp
from jax.experimental import pallas as pl
from jax.experimental.pallas import tpu as pltpu

_B, _N, _DIN, _H, _OUT = 4, 128, 128, 128, 128
_NL, _DE = 4, 4
_NP = _N // 2
_EP = _N * _NP
_HP = 2 * _H

_F32 = jnp.float32
_BF = jnp.bfloat16


def _bd2(w):
    z = jnp.zeros(w.shape, _BF)
    wb = w.astype(_BF)
    return jnp.concatenate(
        [jnp.concatenate([wb, z], axis=1), jnp.concatenate([z, wb], axis=1)], axis=0)


def _gnn_body(x_ref, ea_ref, w1i_ref, b1i_ref, w2i_ref, b2i_ref, *rest):
    h_ref = rest[-1]
    wl_refs = rest[:-1]

    w1i = w1i_ref[...].astype(_BF)
    b1i = b1i_ref[...]
    w2i = w2i_ref[...].astype(_BF)
    b2i = b2i_ref[...]
    layers = []
    for l in range(_NL):
        (we1_ref, be1_ref, we2_ref, be2_ref,
         wn1_ref, bn1_ref, wn2_ref, bn2_ref) = wl_refs[8 * l:8 * (l + 1)]
        we1 = we1_ref[...]
        wn1 = wn1_ref[...]
        be2 = be2_ref[...]
        layers.append((
            we1[:_H].astype(_BF), we1[_H:2 * _H].astype(_BF),
            _bd2(we1[2 * _H:]), be1_ref[...],
            _bd2(we2_ref[...]), jnp.concatenate([be2, be2], axis=1),
            wn1[:_H].astype(_BF), wn1[_H:].astype(_BF), bn1_ref[...],
            wn2_ref[...].astype(_BF), bn2_ref[...]))

    for b in range(_B):
        xb = x_ref[b].astype(_BF)
        h1 = jnp.maximum(
            jnp.dot(xb, w1i, preferred_element_type=_F32) + b1i, 0.0)
        h = jnp.dot(h1.astype(_BF), w2i, preferred_element_type=_F32) + b2i

        ea = ea_ref[b]
        for l in range(_NL):
            (we1a, we1b, we1e_bd, be1, we2_bd, be2_p,
             wn1a, wn1b, bn1, wn2, bn2) = layers[l]
            hb = h.astype(_BF)
            a = jnp.dot(hb, we1a, preferred_element_type=_F32) + be1
            ap = jnp.concatenate([a, a], axis=1)
            c = jnp.dot(hb, we1b, preferred_element_type=_F32)
            cp = jnp.concatenate([c[:_NP], c[_NP:]], axis=1)
            e = jnp.dot(ea, we1e_bd, preferred_element_type=_F32)
            pre = (e.astype(_BF).reshape(_N, _NP, _HP)
                   + ap.astype(_BF).reshape(_N, 1, _HP)
                   + cp.astype(_BF).reshape(1, _NP, _HP))
            m1 = jnp.maximum(pre, _BF(0.0)).reshape(_EP, _HP)
            m2 = jnp.maximum(
                jnp.dot(m1, we2_bd, preferred_element_type=_F32) + be2_p, 0.0)
            ms = m2.reshape(_N, _NP, _HP).sum(axis=1)
            mi = ms[:, :_H] + ms[:, _H:]
            u = jnp.maximum(
                jnp.dot(hb, wn1a, preferred_element_type=_F32)
                + jnp.dot(mi.astype(_BF), wn1b, preferred_element_type=_F32)
                + bn1, 0.0)
            h = h + jnp.dot(u.astype(_BF), wn2, preferred_element_type=_F32) + bn2

        h_ref[b] = h


def _out_body(hf_ref, w1_ref, b1_ref, w2_ref, b2_ref, o_ref):
    h1 = jnp.maximum(
        jnp.dot(hf_ref[...], w1_ref[...].astype(_BF), preferred_element_type=_F32)
        + b1_ref[...], 0.0)
    o_ref[...] = (jnp.dot(h1.astype(_BF), w2_ref[...].astype(_BF),
                          preferred_element_type=_F32)
                  + b2_ref[...])


def kernel(x, edge_attr, key_unused, params):
    p = params
    wl = []
    for l in range(_NL):
        wl += [p['l%d_We1' % l], p['l%d_be1' % l].reshape(1, _H),
               p['l%d_We2' % l], p['l%d_be2' % l].reshape(1, _H),
               p['l%d_Wn1' % l], p['l%d_bn1' % l].reshape(1, _H),
               p['l%d_Wn2' % l], p['l%d_bn2' % l].reshape(1, _H)]
    eap = (edge_attr.astype(_BF)
           .reshape(_B, _N, 2, _NP, _DE)
           .transpose(0, 1, 3, 2, 4)
           .reshape(_B, _EP, 2 * _DE))

    h = pl.pallas_call(
        _gnn_body,
        out_shape=jax.ShapeDtypeStruct((_B, _N, _H), _F32),
    )(x, eap, p['emb_in_W1'], p['emb_in_b1'].reshape(1, _H),
      p['emb_in_W2'], p['emb_in_b2'].reshape(1, _H), *wl)

    hf = h.reshape(_B, _N * _H).astype(_BF)
    out = pl.pallas_call(
        _out_body,
        out_shape=jax.ShapeDtypeStruct((_B, _OUT), _F32),
    )(hf, p['emb_out_W1'], p['emb_out_b1'].reshape(1, _H),
      p['emb_out_W2'], p['emb_out_b2'].reshape(1, _OUT))
    return out


# speedup vs baseline: 1.2820x; 1.0389x over previous
import jax
import jax.numpy as jnp
from jax.experimental import pallas as pl
from jax.experimental.pallas import tpu as pltpu

_B, _N, _DIN, _H, _OUT = 4, 128, 128, 128, 128
_NL, _DE = 4, 4
_NP = _N // 2
_EP = _N * _NP
_HP = 2 * _H

_F32 = jnp.float32
_BF = jnp.bfloat16


def _bd2(w):
    z = jnp.zeros(w.shape, _BF)
    wb = w.astype(_BF)
    return jnp.concatenate(
        [jnp.concatenate([wb, z], axis=1), jnp.concatenate([z, wb], axis=1)], axis=0)


def _gnn_body(x_ref, ea_ref, w1i_ref, b1i_ref, w2i_ref, b2i_ref, *rest):
    h_ref = rest[-1]
    wl_refs = rest[:-1]

    xb = x_ref[0].astype(_BF)
    h1 = jnp.maximum(
        jnp.dot(xb, w1i_ref[...].astype(_BF), preferred_element_type=_F32)
        + b1i_ref[...], 0.0)
    h = (jnp.dot(h1.astype(_BF), w2i_ref[...].astype(_BF), preferred_element_type=_F32)
         + b2i_ref[...])

    ea = ea_ref[0]
    for l in range(_NL):
        (we1_ref, be1_ref, we2_ref, be2_ref,
         wn1_ref, bn1_ref, wn2_ref, bn2_ref) = wl_refs[8 * l:8 * (l + 1)]
        we1 = we1_ref[...]
        hb = h.astype(_BF)
        a = (jnp.dot(hb, we1[:_H].astype(_BF), preferred_element_type=_F32)
             + be1_ref[...])
        ap = jnp.concatenate([a, a], axis=1)
        c = jnp.dot(hb, we1[_H:2 * _H].astype(_BF), preferred_element_type=_F32)
        cp = jnp.concatenate([c[:_NP], c[_NP:]], axis=1)
        e = jnp.dot(ea, _bd2(we1[2 * _H:]), preferred_element_type=_F32)
        pre = (e.astype(_BF).reshape(_N, _NP, _HP)
               + ap.astype(_BF).reshape(_N, 1, _HP)
               + cp.astype(_BF).reshape(1, _NP, _HP))
        m1 = jnp.maximum(pre, _BF(0.0)).reshape(_EP, _HP)
        be2 = be2_ref[...]
        m2 = jnp.maximum(
            jnp.dot(m1, _bd2(we2_ref[...]), preferred_element_type=_F32)
            + jnp.concatenate([be2, be2], axis=1), 0.0)
        ms = m2.reshape(_N, _NP, _HP).sum(axis=1)
        mi = ms[:, :_H] + ms[:, _H:]
        wn1 = wn1_ref[...]
        u = jnp.maximum(
            jnp.dot(hb, wn1[:_H].astype(_BF), preferred_element_type=_F32)
            + jnp.dot(mi.astype(_BF), wn1[_H:].astype(_BF), preferred_element_type=_F32)
            + bn1_ref[...], 0.0)
        h = (h + jnp.dot(u.astype(_BF), wn2_ref[...].astype(_BF),
                         preferred_element_type=_F32)
             + bn2_ref[...])

    h_ref[0] = h


def _out_body(hf_ref, w1_ref, b1_ref, w2_ref, b2_ref, o_ref):
    h1 = jnp.maximum(
        jnp.dot(hf_ref[...], w1_ref[...].astype(_BF), preferred_element_type=_F32)
        + b1_ref[...], 0.0)
    o_ref[...] = (jnp.dot(h1.astype(_BF), w2_ref[...].astype(_BF),
                          preferred_element_type=_F32)
                  + b2_ref[...])


def kernel(x, edge_attr, key_unused, params):
    p = params
    wl = []
    for l in range(_NL):
        wl += [p['l%d_We1' % l], p['l%d_be1' % l].reshape(1, _H),
               p['l%d_We2' % l], p['l%d_be2' % l].reshape(1, _H),
               p['l%d_Wn1' % l], p['l%d_bn1' % l].reshape(1, _H),
               p['l%d_Wn2' % l], p['l%d_bn2' % l].reshape(1, _H)]
    eap = (edge_attr.astype(_BF)
           .reshape(_B, _N, 2, _NP, _DE)
           .transpose(0, 1, 3, 2, 4)
           .reshape(_B, _EP, 2 * _DE))

    _cw = lambda b: (0, 0)
    h = pl.pallas_call(
        _gnn_body,
        grid=(_B,),
        in_specs=[
            pl.BlockSpec((1, _N, _DIN), lambda b: (b, 0, 0)),
            pl.BlockSpec((1, _EP, 2 * _DE), lambda b: (b, 0, 0)),
            pl.BlockSpec((_DIN, _H), _cw),
            pl.BlockSpec((1, _H), _cw),
            pl.BlockSpec((_H, _H), _cw),
            pl.BlockSpec((1, _H), _cw),
        ] + [pl.BlockSpec(w.shape, _cw) for w in wl],
        out_specs=pl.BlockSpec((1, _N, _H), lambda b: (b, 0, 0)),
        out_shape=jax.ShapeDtypeStruct((_B, _N, _H), _F32),
        compiler_params=pltpu.CompilerParams(
            dimension_semantics=("parallel",)),
    )(x, eap, p['emb_in_W1'], p['emb_in_b1'].reshape(1, _H),
      p['emb_in_W2'], p['emb_in_b2'].reshape(1, _H), *wl)

    hf = h.reshape(_B, _N * _H).astype(_BF)
    out = pl.pallas_call(
        _out_body,
        out_shape=jax.ShapeDtypeStruct((_B, _OUT), _F32),
    )(hf, p['emb_out_W1'], p['emb_out_b1'].reshape(1, _H),
      p['emb_out_W2'], p['emb_out_b2'].reshape(1, _OUT))
    return out


# speedup vs baseline: 1.3049x; 1.0178x over previous
import jax
import jax.numpy as jnp
from jax.experimental import pallas as pl
from jax.experimental.pallas import tpu as pltpu

_B, _N, _DIN, _H, _OUT = 4, 128, 128, 128, 128
_NL, _DE = 4, 4
_NP = _N // 2
_EP = _N * _NP
_HP = 2 * _H

_F32 = jnp.float32
_BF = jnp.bfloat16


def _bd2(w):
    z = jnp.zeros(w.shape, _BF)
    wb = w.astype(_BF)
    return jnp.concatenate(
        [jnp.concatenate([wb, z], axis=1), jnp.concatenate([z, wb], axis=1)], axis=0)


def _gnn_body(x_ref, ea_ref, w1i_ref, b1i_ref, w2i_ref, b2i_ref, *rest):
    wo1_ref, bo1_ref, wo2_ref, bo2_ref = rest[-7:-3]
    h_ref, o_ref, wo_scr = rest[-3:]
    wl_refs = rest[:-7]
    b = pl.program_id(0)

    @pl.when(b == 0)
    def _cache_wo():
        wo_scr[...] = wo1_ref[...].astype(_BF)

    xb = x_ref[0].astype(_BF)
    h1 = jnp.maximum(
        jnp.dot(xb, w1i_ref[...].astype(_BF), preferred_element_type=_F32)
        + b1i_ref[...], 0.0)
    h = (jnp.dot(h1.astype(_BF), w2i_ref[...].astype(_BF), preferred_element_type=_F32)
         + b2i_ref[...])

    ea = ea_ref[0]
    for l in range(_NL):
        (we1_ref, be1_ref, we2_ref, be2_ref,
         wn1_ref, bn1_ref, wn2_ref, bn2_ref) = wl_refs[8 * l:8 * (l + 1)]
        we1 = we1_ref[...]
        hb = h.astype(_BF)
        a = (jnp.dot(hb, we1[:_H].astype(_BF), preferred_element_type=_F32)
             + be1_ref[...])
        ap = jnp.concatenate([a, a], axis=1)
        c = jnp.dot(hb, we1[_H:2 * _H].astype(_BF), preferred_element_type=_F32)
        cp = jnp.concatenate([c[:_NP], c[_NP:]], axis=1)
        e = jnp.dot(ea, _bd2(we1[2 * _H:]), preferred_element_type=_F32)
        pre = (e.astype(_BF).reshape(_N, _NP, _HP)
               + ap.astype(_BF).reshape(_N, 1, _HP)
               + cp.astype(_BF).reshape(1, _NP, _HP))
        m1 = jnp.maximum(pre, _BF(0.0)).reshape(_EP, _HP)
        be2 = be2_ref[...]
        m2 = jnp.maximum(
            jnp.dot(m1, _bd2(we2_ref[...]), preferred_element_type=_F32)
            + jnp.concatenate([be2, be2], axis=1), 0.0)
        ms = m2.reshape(_N, _NP, _HP).sum(axis=1)
        mi = ms[:, :_H] + ms[:, _H:]
        wn1 = wn1_ref[...]
        u = jnp.maximum(
            jnp.dot(hb, wn1[:_H].astype(_BF), preferred_element_type=_F32)
            + jnp.dot(mi.astype(_BF), wn1[_H:].astype(_BF), preferred_element_type=_F32)
            + bn1_ref[...], 0.0)
        h = (h + jnp.dot(u.astype(_BF), wn2_ref[...].astype(_BF),
                         preferred_element_type=_F32)
             + bn2_ref[...])

    h_ref[0] = h

    hf = h.astype(_BF).reshape(1, _N * _H)
    o1 = jnp.maximum(
        jnp.dot(hf, wo_scr[...], preferred_element_type=_F32) + bo1_ref[...], 0.0)
    o_ref[0] = (jnp.dot(o1.astype(_BF), wo2_ref[...].astype(_BF),
                        preferred_element_type=_F32)
                + bo2_ref[...])


def kernel(x, edge_attr, key_unused, params):
    p = params
    wl = []
    for l in range(_NL):
        wl += [p['l%d_We1' % l], p['l%d_be1' % l].reshape(1, _H),
               p['l%d_We2' % l], p['l%d_be2' % l].reshape(1, _H),
               p['l%d_Wn1' % l], p['l%d_bn1' % l].reshape(1, _H),
               p['l%d_Wn2' % l], p['l%d_bn2' % l].reshape(1, _H)]
    eap = (edge_attr.astype(_BF)
           .reshape(_B, _N, 2, _NP, _DE)
           .transpose(0, 1, 3, 2, 4)
           .reshape(_B, _EP, 2 * _DE))

    _cw = lambda b: (0, 0)
    _h, out = pl.pallas_call(
        _gnn_body,
        grid=(_B,),
        in_specs=[
            pl.BlockSpec((1, _N, _DIN), lambda b: (b, 0, 0)),
            pl.BlockSpec((1, _EP, 2 * _DE), lambda b: (b, 0, 0)),
            pl.BlockSpec((_DIN, _H), _cw),
            pl.BlockSpec((1, _H), _cw),
            pl.BlockSpec((_H, _H), _cw),
            pl.BlockSpec((1, _H), _cw),
        ] + [pl.BlockSpec(w.shape, _cw) for w in wl] + [
            pl.BlockSpec((_N * _H, _H), _cw),
            pl.BlockSpec((1, _H), _cw),
            pl.BlockSpec((_H, _OUT), _cw),
            pl.BlockSpec((1, _OUT), _cw),
        ],
        out_specs=[
            pl.BlockSpec((1, _N, _H), lambda b: (b, 0, 0)),
            pl.BlockSpec((1, 1, _OUT), lambda b: (b, 0, 0)),
        ],
        out_shape=[
            jax.ShapeDtypeStruct((_B, _N, _H), _F32),
            jax.ShapeDtypeStruct((_B, 1, _OUT), _F32),
        ],
        scratch_shapes=[pltpu.VMEM((_N * _H, _H), _BF)],
        compiler_params=pltpu.CompilerParams(
            dimension_semantics=("arbitrary",)),
    )(x, eap, p['emb_in_W1'], p['emb_in_b1'].reshape(1, _H),
      p['emb_in_W2'], p['emb_in_b2'].reshape(1, _H), *wl,
      p['emb_out_W1'], p['emb_out_b1'].reshape(1, _H),
      p['emb_out_W2'], p['emb_out_b2'].reshape(1, _OUT))
    return out.reshape(_B, _OUT)


# speedup vs baseline: 1.3350x; 1.0231x over previous
import jax
import jax.numpy as jnp
from jax.experimental import pallas as pl
from jax.experimental.pallas import tpu as pltpu

_B, _N, _DIN, _H, _OUT = 4, 128, 128, 128, 128
_NL, _DE = 4, 4
_NP = _N // 2
_EP = _N * _NP
_HP = 2 * _H

_F32 = jnp.float32
_BF = jnp.bfloat16


def _bd2(w):
    z = jnp.zeros(w.shape, _BF)
    wb = w.astype(_BF)
    return jnp.concatenate(
        [jnp.concatenate([wb, z], axis=1), jnp.concatenate([z, wb], axis=1)], axis=0)


def _gnn_body(x_ref, ea_ref, w1i_ref, b1i_ref, w2i_ref, b2i_ref, *rest):
    wo1_ref, bo1_ref, wo2_ref, bo2_ref = rest[-8:-4]
    h_ref, o_ref, wo_scr, hf_scr = rest[-4:]
    wl_refs = rest[:-8]
    b = pl.program_id(0)

    @pl.when(b == 0)
    def _cache_wo():
        wo_scr[...] = wo1_ref[...].astype(_BF)

    xb = x_ref[0].astype(_BF)
    h1 = jnp.maximum(
        jnp.dot(xb, w1i_ref[...].astype(_BF), preferred_element_type=_F32)
        + b1i_ref[...], 0.0)
    h = (jnp.dot(h1.astype(_BF), w2i_ref[...].astype(_BF), preferred_element_type=_F32)
         + b2i_ref[...])

    ea = ea_ref[0]
    for l in range(_NL):
        (we1_ref, be1_ref, we2_ref, be2_ref,
         wn1_ref, bn1_ref, wn2_ref, bn2_ref) = wl_refs[8 * l:8 * (l + 1)]
        we1 = we1_ref[...]
        hb = h.astype(_BF)
        a = (jnp.dot(hb, we1[:_H].astype(_BF), preferred_element_type=_F32)
             + be1_ref[...])
        ap = jnp.concatenate([a, a], axis=1)
        c = jnp.dot(hb, we1[_H:2 * _H].astype(_BF), preferred_element_type=_F32)
        cp = jnp.concatenate([c[:_NP], c[_NP:]], axis=1)
        e = jnp.dot(ea, _bd2(we1[2 * _H:]), preferred_element_type=_F32)
        pre = (e.astype(_BF).reshape(_N, _NP, _HP)
               + ap.astype(_BF).reshape(_N, 1, _HP)
               + cp.astype(_BF).reshape(1, _NP, _HP))
        m1 = jnp.maximum(pre, _BF(0.0)).reshape(_EP, _HP)
        be2 = be2_ref[...]
        m2 = jnp.maximum(
            jnp.dot(m1, _bd2(we2_ref[...]), preferred_element_type=_F32)
            + jnp.concatenate([be2, be2], axis=1), 0.0)
        ms = m2.reshape(_N, _NP, _HP).sum(axis=1)
        mi = ms[:, :_H] + ms[:, _H:]
        wn1 = wn1_ref[...]
        u = jnp.maximum(
            jnp.dot(hb, wn1[:_H].astype(_BF), preferred_element_type=_F32)
            + jnp.dot(mi.astype(_BF), wn1[_H:].astype(_BF), preferred_element_type=_F32)
            + bn1_ref[...], 0.0)
        h = (h + jnp.dot(u.astype(_BF), wn2_ref[...].astype(_BF),
                         preferred_element_type=_F32)
             + bn2_ref[...])

    h_ref[0] = h

    hfb = h.astype(_BF).reshape(1, _N * _H)
    for k in range(_B):
        @pl.when(b == k)
        def _stage(k=k, hfb=hfb):
            hf_scr[k:k + 1, :] = hfb

    @pl.when(b == _B - 1)
    def _emb_out():
        o1 = jnp.maximum(
            jnp.dot(hf_scr[...], wo_scr[...], preferred_element_type=_F32)
            + bo1_ref[...], 0.0)
        o_ref[...] = (jnp.dot(o1.astype(_BF), wo2_ref[...].astype(_BF),
                              preferred_element_type=_F32)
                      + bo2_ref[...])


def kernel(x, edge_attr, key_unused, params):
    p = params
    wl = []
    for l in range(_NL):
        wl += [p['l%d_We1' % l], p['l%d_be1' % l].reshape(1, _H),
               p['l%d_We2' % l], p['l%d_be2' % l].reshape(1, _H),
               p['l%d_Wn1' % l], p['l%d_bn1' % l].reshape(1, _H),
               p['l%d_Wn2' % l], p['l%d_bn2' % l].reshape(1, _H)]
    eap = (edge_attr.astype(_BF)
           .reshape(_B, _N, 2, _NP, _DE)
           .transpose(0, 1, 3, 2, 4)
           .reshape(_B, _EP, 2 * _DE))

    _cw = lambda b: (0, 0)
    _h, out = pl.pallas_call(
        _gnn_body,
        grid=(_B,),
        in_specs=[
            pl.BlockSpec((1, _N, _DIN), lambda b: (b, 0, 0)),
            pl.BlockSpec((1, _EP, 2 * _DE), lambda b: (b, 0, 0)),
            pl.BlockSpec((_DIN, _H), _cw),
            pl.BlockSpec((1, _H), _cw),
            pl.BlockSpec((_H, _H), _cw),
            pl.BlockSpec((1, _H), _cw),
        ] + [pl.BlockSpec(w.shape, _cw) for w in wl] + [
            pl.BlockSpec((_N * _H, _H), _cw),
            pl.BlockSpec((1, _H), _cw),
            pl.BlockSpec((_H, _OUT), _cw),
            pl.BlockSpec((1, _OUT), _cw),
        ],
        out_specs=[
            pl.BlockSpec((1, _N, _H), lambda b: (b, 0, 0)),
            pl.BlockSpec((_B, _OUT), lambda b: (0, 0)),
        ],
        out_shape=[
            jax.ShapeDtypeStruct((_B, _N, _H), _F32),
            jax.ShapeDtypeStruct((_B, _OUT), _F32),
        ],
        scratch_shapes=[pltpu.VMEM((_N * _H, _H), _BF),
                        pltpu.VMEM((_B, _N * _H), _BF)],
        compiler_params=pltpu.CompilerParams(
            dimension_semantics=("arbitrary",)),
    )(x, eap, p['emb_in_W1'], p['emb_in_b1'].reshape(1, _H),
      p['emb_in_W2'], p['emb_in_b2'].reshape(1, _H), *wl,
      p['emb_out_W1'], p['emb_out_b1'].reshape(1, _H),
      p['emb_out_W2'], p['emb_out_b2'].reshape(1, _OUT))
    return out
